# baseline (speedup 1.0000x reference)
import jax
import jax.numpy as jnp
from jax import lax
from jax.experimental import pallas as pl
from jax.experimental.pallas import tpu as pltpu
from jax.experimental.pallas import tpu_sc as plsc

B = 4096
H = 200
D = 64
NROWS = B * H
NW = 32
PER_W = NROWS // NW
C = 128
NCHUNK = PER_W // C
EPS = 1e-8
SQRT_D = 8.0
MAGIC = 0x5F3759DF


def _sc_body(idx_hbm, w_hbm, table_hbm, out_hbm, idx_v, rows_v, w_v, sem):
    wid = lax.axis_index("s") * 2 + lax.axis_index("c")
    pltpu.sync_copy(w_hbm, w_v)
    w_regs = [w_v[pl.ds(k * 16, 16)] for k in range(4)]

    def chunk_body(ci, carry):
        base = wid * PER_W + ci * C
        pltpu.sync_copy(idx_hbm.at[pl.ds(base, C)], idx_v)
        pltpu.async_copy(table_hbm.at[idx_v], rows_v, sem).wait()

        def row_body(r, carry2):
            v = [rows_v[r, pl.ds(k * 16, 16)] for k in range(4)]
            acc = v[0] * v[0] + v[1] * v[1] + v[2] * v[2] + v[3] * v[3]
            iota = lax.iota(jnp.int32, 16)
            for sh in (8, 4, 2, 1):
                perm = jnp.bitwise_xor(iota, sh)
                acc = acc + acc.at[perm].get(mode="promise_in_bounds")
            x = acc + EPS
            bits = lax.bitcast_convert_type(x, jnp.int32)
            y = lax.bitcast_convert_type(
                jnp.full((16,), MAGIC, jnp.int32) - (bits >> 1), jnp.float32)
            y = y * (1.5 - 0.5 * x * y * y)
            y = y * (1.5 - 0.5 * x * y * y)
            y = y * (1.5 - 0.5 * x * y * y)
            s = y * SQRT_D
            for k in range(4):
                rows_v[r, pl.ds(k * 16, 16)] = v[k] * (w_regs[k] * s)
            return carry2

        lax.fori_loop(0, C, row_body, 0)
        pltpu.sync_copy(rows_v, out_hbm.at[pl.ds(base, C)])
        return carry

    lax.fori_loop(0, NCHUNK, chunk_body, 0)


def kernel(input_seq, item_table, rms_weight):
    idx = input_seq.reshape(-1).astype(jnp.int32)
    mesh = plsc.VectorSubcoreMesh(core_axis_name="c", subcore_axis_name="s")
    out = pl.kernel(
        _sc_body,
        out_type=jax.ShapeDtypeStruct((NROWS, D), jnp.float32),
        mesh=mesh,
        compiler_params=pltpu.CompilerParams(use_tc_tiling_on_sc=False),
        scratch_types=[
            pltpu.VMEM((C,), jnp.int32),
            pltpu.VMEM((C, D), jnp.float32),
            pltpu.VMEM((D,), jnp.float32),
            pltpu.SemaphoreType.DMA,
        ],
    )(idx, rms_weight, item_table)
    return out.reshape(B, H, D)


# speedup vs baseline: 1.7753x; 1.7753x over previous
import jax
import jax.numpy as jnp
from jax import lax
from jax.experimental import pallas as pl
from jax.experimental.pallas import tpu as pltpu
from jax.experimental.pallas import tpu_sc as plsc

B = 4096
H = 200
D = 64
NROWS = B * H
NW = 32
PER_W = NROWS // NW
C = 256
NCHUNK = PER_W // C
NBUF = 4
AHEAD = 2
U = 4
EPS = 1e-8
SQRT_D = 8.0
MAGIC = 0x5F3759DF


def _sc_body(idx_hbm, w_hbm, table_hbm, out_hbm,
             idx_all, rows, w_v, sem_g, sem_o):
    wid = lax.axis_index("s") * 2 + lax.axis_index("c")
    base_w = wid * PER_W
    pltpu.sync_copy(w_hbm, w_v)
    pltpu.sync_copy(idx_hbm.at[pl.ds(base_w, PER_W)], idx_all)
    w_regs = [w_v[pl.ds(k * 16, 16)] for k in range(4)]

    def start_gather(ci, p):
        pltpu.async_copy(
            table_hbm.at[idx_all.at[pl.ds(ci * C, C)]],
            rows.at[p], sem_g.at[p])

    for b in range(AHEAD):
        start_gather(b, b)

    def compute_chunk(p):
        def row_body(i, carry):
            r0 = i * U
            vs = []
            xs = []
            for u in range(U):
                v = [rows[p, r0 + u, pl.ds(k * 16, 16)] for k in range(4)]
                acc = v[0] * v[0] + v[1] * v[1] + v[2] * v[2] + v[3] * v[3]
                vs.append(v)
                xs.append(acc)
            iota = lax.iota(jnp.int32, 16)
            for sh in (8, 4, 2, 1):
                perm = jnp.bitwise_xor(iota, sh)
                xs = [a + a.at[perm].get(mode="promise_in_bounds")
                      for a in xs]
            for u in range(U):
                x = xs[u] + EPS
                bits = lax.bitcast_convert_type(x, jnp.int32)
                y = lax.bitcast_convert_type(
                    jnp.full((16,), MAGIC, jnp.int32) - (bits >> 1),
                    jnp.float32)
                y = y * (1.5 - 0.5 * x * y * y)
                y = y * (1.5 - 0.5 * x * y * y)
                s = y * SQRT_D
                for k in range(4):
                    rows[p, r0 + u, pl.ds(k * 16, 16)] = (
                        vs[u][k] * (w_regs[k] * s))
            return carry
        lax.fori_loop(0, C // U, row_body, 0, unroll=1)

    def chunk_body(ci, carry):
        p = lax.rem(ci, NBUF)
        nxt = ci + AHEAD

        @pl.when(nxt < NCHUNK)
        def _():
            q = lax.rem(nxt, NBUF)

            @pl.when(ci >= NBUF - AHEAD)
            def _():
                pltpu.make_async_copy(rows.at[q], out_hbm.at[pl.ds(0, C)],
                                      sem_o.at[q]).wait()
            start_gather(nxt, q)

        pltpu.make_async_copy(table_hbm.at[idx_all.at[pl.ds(0, C)]],
                              rows.at[p], sem_g.at[p]).wait()
        compute_chunk(p)
        pltpu.async_copy(rows.at[p], out_hbm.at[pl.ds(base_w + ci * C, C)],
                         sem_o.at[p])
        return carry

    lax.fori_loop(0, NCHUNK, chunk_body, 0)
    for p in range(NBUF):
        pltpu.make_async_copy(rows.at[p], out_hbm.at[pl.ds(0, C)],
                              sem_o.at[p]).wait()


def kernel(input_seq, item_table, rms_weight):
    idx = input_seq.reshape(-1).astype(jnp.int32)
    mesh = plsc.VectorSubcoreMesh(core_axis_name="c", subcore_axis_name="s")
    out = pl.kernel(
        _sc_body,
        out_type=jax.ShapeDtypeStruct((NROWS, D), jnp.float32),
        mesh=mesh,
        compiler_params=pltpu.CompilerParams(use_tc_tiling_on_sc=False),
        scratch_types=[
            pltpu.VMEM((PER_W,), jnp.int32),
            pltpu.VMEM((NBUF, C, D), jnp.float32),
            pltpu.VMEM((D,), jnp.float32),
            pltpu.SemaphoreType.DMA((NBUF,)),
            pltpu.SemaphoreType.DMA((NBUF,)),
        ],
    )(idx, rms_weight, item_table)
    return out.reshape(B, H, D)
